# baseline (speedup 1.0000x reference)
import functools

import jax
import jax.numpy as jnp
from jax import lax
from jax.experimental import pallas as pl
from jax.experimental.pallas import tpu as pltpu
from jax.experimental.pallas import tpu_sc as plsc

N_T = 10000
N_N = 10000
E = 320000
D = 128

NC = 2
NS = 16
NW = NC * NS

CHUNK = 128
NCH0 = 58
NCH1 = 100
NCHM = max(NCH0, NCH1)

N_ROWS = 10000
N_PAD = 10112
JUNK = N_ROWS
RPT = N_PAD // NS


def _split_cores(flat, fill):
    n0 = NS * NCH0 * CHUNK
    a0 = flat[:n0].reshape(NS, NCH0, CHUNK)
    a0 = jnp.pad(a0, ((0, 0), (0, NCHM - NCH0), (0, 0)), constant_values=fill)
    a1 = flat[n0:].reshape(NS, NCH1, CHUNK)
    a1 = jnp.pad(a1, ((0, 0), (0, NCHM - NCH1), (0, 0)), constant_values=fill)
    return jnp.concatenate([a0, a1], axis=0)


def _prep_edges(edges):
    src = edges[0].astype(jnp.int32)
    dst = edges[1].astype(jnp.int32)
    pad = NS * (NCH0 + NCH1) * CHUNK - E
    src = jnp.concatenate([src, jnp.zeros((pad,), jnp.int32)])
    dst = jnp.concatenate([dst, jnp.full((pad,), JUNK, jnp.int32)])
    return _split_cores(src, 0), _split_cores(dst, JUNK)


@functools.partial(
    pl.kernel,
    out_type=jax.ShapeDtypeStruct((NC, N_PAD, D), jnp.float32),
    mesh=plsc.VectorSubcoreMesh(core_axis_name="c", subcore_axis_name="s"),
    scratch_types=[
        pltpu.VMEM_SHARED((N_PAD, D), jnp.float32),
        pltpu.VMEM((NCHM, CHUNK), jnp.int32),
        pltpu.VMEM((NCHM, CHUNK), jnp.int32),
        pltpu.VMEM((CHUNK, D), jnp.float32),
    ],
)
def _segment_sum_sc(table_hbm, src_hbm, dst_hbm, zeros_hbm, out_hbm,
                    acc, dst_v, sidx, buf):
    c = lax.axis_index("c")
    s = lax.axis_index("s")
    wid = c * NS + s

    pltpu.sync_copy(zeros_hbm, buf)
    base = s * RPT
    for k in range(RPT // CHUNK):
        pltpu.sync_copy(buf, acc.at[pl.ds(base + k * CHUNK, CHUNK)])
    rem = RPT % CHUNK
    if rem:
        pltpu.sync_copy(buf.at[pl.ds(0, rem)],
                        acc.at[pl.ds(base + RPT - rem, rem)])
    pltpu.sync_copy(dst_hbm.at[wid], dst_v)
    plsc.subcore_barrier()

    pltpu.sync_copy(src_hbm.at[wid], sidx)

    def body(j, carry):
        pltpu.sync_copy(table_hbm.at[sidx.at[j]], buf)
        pltpu.sync_copy(buf, acc.at[dst_v.at[j]], add=True)
        return carry

    nch = jnp.where(c == 0, NCH0, NCH1)
    lax.fori_loop(0, nch, body, 0)
    plsc.subcore_barrier()

    for k in range(RPT // CHUNK):
        pltpu.sync_copy(acc.at[pl.ds(base + k * CHUNK, CHUNK)], buf)
        pltpu.sync_copy(buf, out_hbm.at[c, pl.ds(base + k * CHUNK, CHUNK)])
    if rem:
        pltpu.sync_copy(acc.at[pl.ds(base + RPT - rem, rem)],
                        buf.at[pl.ds(0, rem)])
        pltpu.sync_copy(buf.at[pl.ds(0, rem)],
                        out_hbm.at[c, pl.ds(base + RPT - rem, rem)])


BLK = 1000


def _mlp1_body(eps_ref, x_ref, a0_ref, a1_ref, w_ref, b_ref, o_ref):
    h = (1.0 + eps_ref[0]) * x_ref[...] + a0_ref[0] + a1_ref[0]
    o_ref[...] = jnp.dot(h, w_ref[...], preferred_element_type=jnp.float32,
                         precision=lax.Precision.HIGHEST) + b_ref[...]


def _mlp1(eps, x, a, w, b):
    row = pl.BlockSpec((BLK, D), lambda i: (i, 0))
    part0 = pl.BlockSpec((1, BLK, D), lambda i: (0, i, 0))
    part1 = pl.BlockSpec((1, BLK, D), lambda i: (1, i, 0))
    full = pl.BlockSpec((D, D), lambda i: (0, 0))
    vec = pl.BlockSpec((1, D), lambda i: (0, 0))
    return pl.pallas_call(
        _mlp1_body,
        grid=(N_ROWS // BLK,),
        in_specs=[pl.BlockSpec(memory_space=pltpu.SMEM),
                  row, part0, part1, full, vec],
        out_specs=row,
        out_shape=jax.ShapeDtypeStruct((N_ROWS, D), jnp.float32),
    )(eps, x, a, a, w, b.reshape(1, D))


def _mlp2_body(eps_ref, x_ref, a0_ref, a1_ref, wp_ref, bp_ref, wr_ref, br_ref,
               o_ref):
    proj = jnp.dot(x_ref[...], wp_ref[...], preferred_element_type=jnp.float32,
                   precision=lax.Precision.HIGHEST) + bp_ref[...]
    h = (1.0 + eps_ref[0]) * proj + a0_ref[0] + a1_ref[0]
    o_ref[...] = jnp.dot(h, wr_ref[...], preferred_element_type=jnp.float32,
                         precision=lax.Precision.HIGHEST) + br_ref[...]


def _mlp2(eps, x, a, wp, bp, wr, br):
    row = pl.BlockSpec((BLK, D), lambda i: (i, 0))
    part0 = pl.BlockSpec((1, BLK, D), lambda i: (0, i, 0))
    part1 = pl.BlockSpec((1, BLK, D), lambda i: (1, i, 0))
    full = pl.BlockSpec((D, D), lambda i: (0, 0))
    vec = pl.BlockSpec((1, D), lambda i: (0, 0))
    return pl.pallas_call(
        _mlp2_body,
        grid=(N_ROWS // BLK,),
        in_specs=[pl.BlockSpec(memory_space=pltpu.SMEM),
                  row, part0, part1, full, vec, full, vec],
        out_specs=row,
        out_shape=jax.ShapeDtypeStruct((N_ROWS, D), jnp.float32),
    )(eps, x, a, a, wp, bp.reshape(1, D), wr, br.reshape(1, D))


def kernel(x_target, x_neighbor, edge_go, edge_ret, W_proj, b_proj, W_go, b_go,
           W_ret, b_ret, eps_go, eps_ret):
    src_go, dst_go = _prep_edges(edge_go)
    src_ret, dst_ret = _prep_edges(edge_ret)
    zeros = jnp.zeros((CHUNK, D), jnp.float32)

    a_n = _segment_sum_sc(x_target, src_go, dst_go, zeros)
    h_n = _mlp1(eps_go, x_neighbor, a_n, W_go, b_go)

    a_t = _segment_sum_sc(h_n, src_ret, dst_ret, zeros)
    h_t = _mlp2(eps_ret, x_target, a_t, W_proj, b_proj, W_ret, b_ret)
    return (h_t, h_n)


# speedup vs baseline: 1.2249x; 1.2249x over previous
import functools

import jax
import jax.numpy as jnp
from jax import lax
from jax.experimental import pallas as pl
from jax.experimental.pallas import tpu as pltpu
from jax.experimental.pallas import tpu_sc as plsc

N_T = 10000
N_N = 10000
E = 320000
D = 128

NC = 2
NS = 16
NW = NC * NS

CHUNK = 128
NCH0 = 100
NCH1 = 58
NCHM = max(NCH0, NCH1)

N_ROWS = 10000
N_PAD = 10112
JUNK = N_ROWS
RPT = N_PAD // NS


def _split_cores(flat, fill):
    n0 = NS * NCH0 * CHUNK
    a0 = flat[:n0].reshape(NS, NCH0, CHUNK)
    a0 = jnp.pad(a0, ((0, 0), (0, NCHM - NCH0), (0, 0)), constant_values=fill)
    a1 = flat[n0:].reshape(NS, NCH1, CHUNK)
    a1 = jnp.pad(a1, ((0, 0), (0, NCHM - NCH1), (0, 0)), constant_values=fill)
    return jnp.concatenate([a0, a1], axis=0)


def _prep_edges(edges):
    src = edges[0].astype(jnp.int32)
    dst = edges[1].astype(jnp.int32)
    pad = NS * (NCH0 + NCH1) * CHUNK - E
    src = jnp.concatenate([src, jnp.zeros((pad,), jnp.int32)])
    dst = jnp.concatenate([dst, jnp.full((pad,), JUNK, jnp.int32)])
    return _split_cores(src, 0), _split_cores(dst, JUNK)


@functools.partial(
    pl.kernel,
    out_type=jax.ShapeDtypeStruct((NC, N_PAD, D), jnp.float32),
    mesh=plsc.VectorSubcoreMesh(core_axis_name="c", subcore_axis_name="s"),
    scratch_types=[
        pltpu.VMEM_SHARED((N_PAD, D), jnp.float32),
        pltpu.VMEM((NCHM, CHUNK), jnp.int32),
        pltpu.VMEM((NCHM, CHUNK), jnp.int32),
        pltpu.VMEM((CHUNK, D), jnp.float32),
    ],
)
def _segment_sum_sc(table_hbm, src_hbm, dst_hbm, zeros_hbm, out_hbm,
                    acc, dst_v, sidx, buf):
    c = lax.axis_index("c")
    s = lax.axis_index("s")
    wid = c * NS + s

    pltpu.sync_copy(zeros_hbm, buf)
    base = s * RPT
    for k in range(RPT // CHUNK):
        pltpu.sync_copy(buf, acc.at[pl.ds(base + k * CHUNK, CHUNK)])
    rem = RPT % CHUNK
    if rem:
        pltpu.sync_copy(buf.at[pl.ds(0, rem)],
                        acc.at[pl.ds(base + RPT - rem, rem)])
    pltpu.sync_copy(dst_hbm.at[wid], dst_v)
    plsc.subcore_barrier()

    pltpu.sync_copy(src_hbm.at[wid], sidx)

    def body(j, carry):
        pltpu.sync_copy(table_hbm.at[sidx.at[j]], buf)
        pltpu.sync_copy(buf, acc.at[dst_v.at[j]], add=True)
        return carry

    nch = jnp.where(c == 0, NCH0, NCH1)
    lax.fori_loop(0, nch, body, 0)
    plsc.subcore_barrier()

    for k in range(RPT // CHUNK):
        pltpu.sync_copy(acc.at[pl.ds(base + k * CHUNK, CHUNK)], buf)
        pltpu.sync_copy(buf, out_hbm.at[c, pl.ds(base + k * CHUNK, CHUNK)])
    if rem:
        pltpu.sync_copy(acc.at[pl.ds(base + RPT - rem, rem)],
                        buf.at[pl.ds(0, rem)])
        pltpu.sync_copy(buf.at[pl.ds(0, rem)],
                        out_hbm.at[c, pl.ds(base + RPT - rem, rem)])


BLK = 1000


def _mlp1_body(eps_ref, x_ref, a0_ref, a1_ref, w_ref, b_ref, o_ref):
    h = (1.0 + eps_ref[0]) * x_ref[...] + a0_ref[0] + a1_ref[0]
    o_ref[...] = jnp.dot(h, w_ref[...], preferred_element_type=jnp.float32,
                         precision=lax.Precision.HIGHEST) + b_ref[...]


def _mlp1(eps, x, a, w, b):
    row = pl.BlockSpec((BLK, D), lambda i: (i, 0))
    part0 = pl.BlockSpec((1, BLK, D), lambda i: (0, i, 0))
    part1 = pl.BlockSpec((1, BLK, D), lambda i: (1, i, 0))
    full = pl.BlockSpec((D, D), lambda i: (0, 0))
    vec = pl.BlockSpec((1, D), lambda i: (0, 0))
    return pl.pallas_call(
        _mlp1_body,
        grid=(N_ROWS // BLK,),
        in_specs=[pl.BlockSpec(memory_space=pltpu.SMEM),
                  row, part0, part1, full, vec],
        out_specs=row,
        out_shape=jax.ShapeDtypeStruct((N_ROWS, D), jnp.float32),
    )(eps, x, a, a, w, b.reshape(1, D))


def _mlp2_body(eps_ref, x_ref, a0_ref, a1_ref, wp_ref, bp_ref, wr_ref, br_ref,
               o_ref):
    proj = jnp.dot(x_ref[...], wp_ref[...], preferred_element_type=jnp.float32,
                   precision=lax.Precision.HIGHEST) + bp_ref[...]
    h = (1.0 + eps_ref[0]) * proj + a0_ref[0] + a1_ref[0]
    o_ref[...] = jnp.dot(h, wr_ref[...], preferred_element_type=jnp.float32,
                         precision=lax.Precision.HIGHEST) + br_ref[...]


def _mlp2(eps, x, a, wp, bp, wr, br):
    row = pl.BlockSpec((BLK, D), lambda i: (i, 0))
    part0 = pl.BlockSpec((1, BLK, D), lambda i: (0, i, 0))
    part1 = pl.BlockSpec((1, BLK, D), lambda i: (1, i, 0))
    full = pl.BlockSpec((D, D), lambda i: (0, 0))
    vec = pl.BlockSpec((1, D), lambda i: (0, 0))
    return pl.pallas_call(
        _mlp2_body,
        grid=(N_ROWS // BLK,),
        in_specs=[pl.BlockSpec(memory_space=pltpu.SMEM),
                  row, part0, part1, full, vec, full, vec],
        out_specs=row,
        out_shape=jax.ShapeDtypeStruct((N_ROWS, D), jnp.float32),
    )(eps, x, a, a, wp, bp.reshape(1, D), wr, br.reshape(1, D))


def kernel(x_target, x_neighbor, edge_go, edge_ret, W_proj, b_proj, W_go, b_go,
           W_ret, b_ret, eps_go, eps_ret):
    src_go, dst_go = _prep_edges(edge_go)
    src_ret, dst_ret = _prep_edges(edge_ret)
    zeros = jnp.zeros((CHUNK, D), jnp.float32)

    a_n = _segment_sum_sc(x_target, src_go, dst_go, zeros)
    h_n = _mlp1(eps_go, x_neighbor, a_n, W_go, b_go)

    a_t = _segment_sum_sc(h_n, src_ret, dst_ret, zeros)
    h_t = _mlp2(eps_ret, x_target, a_t, W_proj, b_proj, W_ret, b_ret)
    return (h_t, h_n)


# speedup vs baseline: 1.3429x; 1.0963x over previous
import functools

import jax
import jax.numpy as jnp
from jax import lax
from jax.experimental import pallas as pl
from jax.experimental.pallas import tpu as pltpu
from jax.experimental.pallas import tpu_sc as plsc

N_T = 10000
N_N = 10000
E = 320000
D = 128

NC = 2
NS = 16
NW = NC * NS

CHUNK = 128
NCH0 = 112
NCH1 = 46
NCHM = max(NCH0, NCH1)

N_ROWS = 10000
N_PAD = 10112
JUNK = N_ROWS
RPT = N_PAD // NS


def _split_cores(flat, fill):
    n0 = NS * NCH0 * CHUNK
    a0 = flat[:n0].reshape(NS, NCH0, CHUNK)
    a0 = jnp.pad(a0, ((0, 0), (0, NCHM - NCH0), (0, 0)), constant_values=fill)
    a1 = flat[n0:].reshape(NS, NCH1, CHUNK)
    a1 = jnp.pad(a1, ((0, 0), (0, NCHM - NCH1), (0, 0)), constant_values=fill)
    return jnp.concatenate([a0, a1], axis=0)


def _prep_edges(edges):
    src = edges[0].astype(jnp.int32)
    dst = edges[1].astype(jnp.int32)
    pad = NS * (NCH0 + NCH1) * CHUNK - E
    src = jnp.concatenate([src, jnp.zeros((pad,), jnp.int32)])
    dst = jnp.concatenate([dst, jnp.full((pad,), JUNK, jnp.int32)])
    return _split_cores(src, 0), _split_cores(dst, JUNK)


@functools.partial(
    pl.kernel,
    out_type=jax.ShapeDtypeStruct((NC, N_PAD, D), jnp.float32),
    mesh=plsc.VectorSubcoreMesh(core_axis_name="c", subcore_axis_name="s"),
    scratch_types=[
        pltpu.VMEM_SHARED((N_PAD, D), jnp.float32),
        pltpu.VMEM((NCHM, CHUNK), jnp.int32),
        pltpu.VMEM((NCHM, CHUNK), jnp.int32),
        pltpu.VMEM((CHUNK, D), jnp.float32),
    ],
)
def _segment_sum_sc(table_hbm, src_hbm, dst_hbm, zeros_hbm, out_hbm,
                    acc, dst_v, sidx, buf):
    c = lax.axis_index("c")
    s = lax.axis_index("s")
    wid = c * NS + s

    pltpu.sync_copy(zeros_hbm, buf)
    base = s * RPT
    for k in range(RPT // CHUNK):
        pltpu.sync_copy(buf, acc.at[pl.ds(base + k * CHUNK, CHUNK)])
    rem = RPT % CHUNK
    if rem:
        pltpu.sync_copy(buf.at[pl.ds(0, rem)],
                        acc.at[pl.ds(base + RPT - rem, rem)])
    pltpu.sync_copy(dst_hbm.at[wid], dst_v)
    plsc.subcore_barrier()

    pltpu.sync_copy(src_hbm.at[wid], sidx)

    def body(j, carry):
        pltpu.sync_copy(table_hbm.at[sidx.at[j]], buf)
        pltpu.sync_copy(buf, acc.at[dst_v.at[j]], add=True)
        return carry

    nch = jnp.where(c == 0, NCH0, NCH1)
    lax.fori_loop(0, nch, body, 0)
    plsc.subcore_barrier()

    for k in range(RPT // CHUNK):
        pltpu.sync_copy(acc.at[pl.ds(base + k * CHUNK, CHUNK)], buf)
        pltpu.sync_copy(buf, out_hbm.at[c, pl.ds(base + k * CHUNK, CHUNK)])
    if rem:
        pltpu.sync_copy(acc.at[pl.ds(base + RPT - rem, rem)],
                        buf.at[pl.ds(0, rem)])
        pltpu.sync_copy(buf.at[pl.ds(0, rem)],
                        out_hbm.at[c, pl.ds(base + RPT - rem, rem)])


BLK = 1000


def _mlp1_body(eps_ref, x_ref, a0_ref, a1_ref, w_ref, b_ref, o_ref):
    h = (1.0 + eps_ref[0]) * x_ref[...] + a0_ref[0] + a1_ref[0]
    o_ref[...] = jnp.dot(h, w_ref[...], preferred_element_type=jnp.float32,
                         precision=lax.Precision.HIGHEST) + b_ref[...]


def _mlp1(eps, x, a, w, b):
    row = pl.BlockSpec((BLK, D), lambda i: (i, 0))
    part0 = pl.BlockSpec((1, BLK, D), lambda i: (0, i, 0))
    part1 = pl.BlockSpec((1, BLK, D), lambda i: (1, i, 0))
    full = pl.BlockSpec((D, D), lambda i: (0, 0))
    vec = pl.BlockSpec((1, D), lambda i: (0, 0))
    return pl.pallas_call(
        _mlp1_body,
        grid=(N_ROWS // BLK,),
        in_specs=[pl.BlockSpec(memory_space=pltpu.SMEM),
                  row, part0, part1, full, vec],
        out_specs=row,
        out_shape=jax.ShapeDtypeStruct((N_ROWS, D), jnp.float32),
    )(eps, x, a, a, w, b.reshape(1, D))


def _mlp2_body(eps_ref, x_ref, a0_ref, a1_ref, wp_ref, bp_ref, wr_ref, br_ref,
               o_ref):
    proj = jnp.dot(x_ref[...], wp_ref[...], preferred_element_type=jnp.float32,
                   precision=lax.Precision.HIGHEST) + bp_ref[...]
    h = (1.0 + eps_ref[0]) * proj + a0_ref[0] + a1_ref[0]
    o_ref[...] = jnp.dot(h, wr_ref[...], preferred_element_type=jnp.float32,
                         precision=lax.Precision.HIGHEST) + br_ref[...]


def _mlp2(eps, x, a, wp, bp, wr, br):
    row = pl.BlockSpec((BLK, D), lambda i: (i, 0))
    part0 = pl.BlockSpec((1, BLK, D), lambda i: (0, i, 0))
    part1 = pl.BlockSpec((1, BLK, D), lambda i: (1, i, 0))
    full = pl.BlockSpec((D, D), lambda i: (0, 0))
    vec = pl.BlockSpec((1, D), lambda i: (0, 0))
    return pl.pallas_call(
        _mlp2_body,
        grid=(N_ROWS // BLK,),
        in_specs=[pl.BlockSpec(memory_space=pltpu.SMEM),
                  row, part0, part1, full, vec, full, vec],
        out_specs=row,
        out_shape=jax.ShapeDtypeStruct((N_ROWS, D), jnp.float32),
    )(eps, x, a, a, wp, bp.reshape(1, D), wr, br.reshape(1, D))


def kernel(x_target, x_neighbor, edge_go, edge_ret, W_proj, b_proj, W_go, b_go,
           W_ret, b_ret, eps_go, eps_ret):
    src_go, dst_go = _prep_edges(edge_go)
    src_ret, dst_ret = _prep_edges(edge_ret)
    zeros = jnp.zeros((CHUNK, D), jnp.float32)

    a_n = _segment_sum_sc(x_target, src_go, dst_go, zeros)
    h_n = _mlp1(eps_go, x_neighbor, a_n, W_go, b_go)

    a_t = _segment_sum_sc(h_n, src_ret, dst_ret, zeros)
    h_t = _mlp2(eps_ret, x_target, a_t, W_proj, b_proj, W_ret, b_ret)
    return (h_t, h_n)
